# Initial kernel scaffold; baseline (speedup 1.0000x reference)
import jax
import jax.numpy as jnp
from jax.experimental import pallas as pl


def kernel(scalar_features, vector_features, edge_index, edge_rbf, edge_vector, W1, b1, W2, b2):
    raise NotImplementedError("write your pallas kernel here")


# speedup vs baseline: 13.1765x; 13.1765x over previous
import functools

import jax
import jax.numpy as jnp
from jax import lax
from jax.experimental import pallas as pl
from jax.experimental.pallas import tpu as pltpu
from jax.experimental.pallas import tpu_sc as plsc

N = 10000
E = 160000
H = 128
NRBF = 20
LANES = 16

CHUNK = 128
NCHUNKS = E // CHUNK
NW = 32
NTILES = 16
NPAD = 10112
ROWS_PER_TILE = NPAD // NTILES

BE = 640
BN = 400


def _mlp_body(rbf_ref, w1_ref, b1_ref, w2_ref, b2_ref, fs_ref, f1_ref, f2_ref):
    x = jnp.dot(rbf_ref[...], w1_ref[...],
                preferred_element_type=jnp.float32) + b1_ref[...]
    h = x * jax.nn.sigmoid(x)
    fw = jnp.dot(h, w2_ref[...], preferred_element_type=jnp.float32)
    b2 = b2_ref[...]
    fs_ref[...] = fw[:, 0:H] + b2[:, 0:H]
    f1_ref[...] = fw[:, H:2 * H] + b2[:, H:2 * H]
    f2_ref[...] = fw[:, 2 * H:3 * H] + b2[:, 2 * H:3 * H]


def _edge_mlp(rbf, W1, b1, W2, b2):
    out = jax.ShapeDtypeStruct((E, H), jnp.float32)
    return pl.pallas_call(
        _mlp_body,
        grid=(E // BE,),
        in_specs=[
            pl.BlockSpec((BE, NRBF), lambda i: (i, 0)),
            pl.BlockSpec((NRBF, H), lambda i: (0, 0)),
            pl.BlockSpec((1, H), lambda i: (0, 0)),
            pl.BlockSpec((H, 3 * H), lambda i: (0, 0)),
            pl.BlockSpec((1, 3 * H), lambda i: (0, 0)),
        ],
        out_specs=[
            pl.BlockSpec((BE, H), lambda i: (i, 0)),
            pl.BlockSpec((BE, H), lambda i: (i, 0)),
            pl.BlockSpec((BE, H), lambda i: (i, 0)),
        ],
        out_shape=[out, out, out],
    )(rbf, W1, b1.reshape(1, H), W2, b2.reshape(1, 3 * H))


def _combine_body(sf_ref, vf_ref, ps_ref, p0_ref, p1_ref, p2_ref,
                  so_ref, vo_ref):
    so_ref[...] = sf_ref[...] + ps_ref[0] + ps_ref[1]
    vo_ref[:, 0, :] = vf_ref[:, 0, :] + p0_ref[0] + p0_ref[1]
    vo_ref[:, 1, :] = vf_ref[:, 1, :] + p1_ref[0] + p1_ref[1]
    vo_ref[:, 2, :] = vf_ref[:, 2, :] + p2_ref[0] + p2_ref[1]


def _combine(sf, vf, ps, p0, p1, p2):
    part_spec = pl.BlockSpec((2, BN, H), lambda i: (0, i, 0))
    return pl.pallas_call(
        _combine_body,
        grid=(N // BN,),
        in_specs=[
            pl.BlockSpec((BN, H), lambda i: (i, 0)),
            pl.BlockSpec((BN, 3, H), lambda i: (i, 0, 0)),
            part_spec, part_spec, part_spec, part_spec,
        ],
        out_specs=[
            pl.BlockSpec((BN, H), lambda i: (i, 0)),
            pl.BlockSpec((BN, 3, H), lambda i: (i, 0, 0)),
        ],
        out_shape=[
            jax.ShapeDtypeStruct((N, H), jnp.float32),
            jax.ShapeDtypeStruct((N, 3, H), jnp.float32),
        ],
    )(sf, vf, ps, p0, p1, p2)


def _num_chunks(wid):
    base = NCHUNKS // NW
    return lax.select(wid < NCHUNKS % NW, base + 1, base)


def _make_scalar_pass():
    mesh = plsc.VectorSubcoreMesh(core_axis_name="c", subcore_axis_name="s")

    @functools.partial(
        pl.kernel, mesh=mesh,
        out_type=jax.ShapeDtypeStruct((2, NPAD, H), jnp.float32),
        scratch_types=[
            pltpu.VMEM((CHUNK,), jnp.int32),
            pltpu.VMEM((CHUNK,), jnp.int32),
            pltpu.VMEM((CHUNK, H), jnp.float32),
            pltpu.VMEM((CHUNK, H), jnp.float32),
            pltpu.VMEM_SHARED((NPAD, H), jnp.float32),
            pltpu.SemaphoreType.DMA,
        ],
    )
    def scalar_pass(tbl, srci, dsti, fs, zz, out, sidx, didx, rows, frows,
                    acc, sem):
        cid = lax.axis_index("c")
        sid = lax.axis_index("s")
        wid = sid * 2 + cid
        r0 = sid * ROWS_PER_TILE
        pltpu.sync_copy(zz.at[pl.ds(r0, ROWS_PER_TILE)],
                        acc.at[pl.ds(r0, ROWS_PER_TILE)])
        plsc.subcore_barrier()

        def chunk_body(i, carry):
            c0 = (wid + i * NW) * CHUNK
            pltpu.sync_copy(srci.at[pl.ds(c0, CHUNK)], sidx)
            pltpu.sync_copy(dsti.at[pl.ds(c0, CHUNK)], didx)
            pltpu.async_copy(tbl.at[sidx], rows, sem).wait()
            pltpu.sync_copy(fs.at[pl.ds(c0, CHUNK)], frows)

            def e_body(e, _):
                for hh in range(H // LANES):
                    s = pl.ds(hh * LANES, LANES)
                    rows[e, s] = rows[e, s] * frows[e, s]
                return 0

            lax.fori_loop(0, CHUNK, e_body, 0)
            pltpu.sync_copy(rows, acc.at[didx], add=True)
            return 0

        lax.fori_loop(0, _num_chunks(wid), chunk_body, 0)
        plsc.subcore_barrier()
        pltpu.sync_copy(acc.at[pl.ds(r0, ROWS_PER_TILE)],
                        out.at[cid, pl.ds(r0, ROWS_PER_TILE)])

    return scalar_pass


def _make_vector_pass():
    mesh = plsc.VectorSubcoreMesh(core_axis_name="c", subcore_axis_name="s")

    @functools.partial(
        pl.kernel, mesh=mesh,
        out_type=jax.ShapeDtypeStruct((2, NPAD, H), jnp.float32),
        scratch_types=[
            pltpu.VMEM((CHUNK,), jnp.int32),
            pltpu.VMEM((CHUNK,), jnp.int32),
            pltpu.VMEM((CHUNK, H), jnp.float32),
            pltpu.VMEM((CHUNK, H), jnp.float32),
            pltpu.VMEM((CHUNK, H), jnp.float32),
            pltpu.VMEM((CHUNK + LANES,), jnp.float32),
            pltpu.VMEM_SHARED((NPAD, H), jnp.float32),
            pltpu.SemaphoreType.DMA,
        ],
    )
    def vector_pass(tbl, srci, dsti, f1, f2, ev, zz, out, sidx, didx, rows,
                    f1rows, f2rows, evv, acc, sem):
        cid = lax.axis_index("c")
        sid = lax.axis_index("s")
        wid = sid * 2 + cid
        r0 = sid * ROWS_PER_TILE
        pltpu.sync_copy(zz.at[pl.ds(r0, ROWS_PER_TILE)],
                        acc.at[pl.ds(r0, ROWS_PER_TILE)])
        plsc.subcore_barrier()

        def chunk_body(i, carry):
            c0 = (wid + i * NW) * CHUNK
            pltpu.sync_copy(srci.at[pl.ds(c0, CHUNK)], sidx)
            pltpu.sync_copy(dsti.at[pl.ds(c0, CHUNK)], didx)
            pltpu.async_copy(tbl.at[sidx], rows, sem).wait()
            pltpu.sync_copy(f1.at[pl.ds(c0, CHUNK)], f1rows)
            pltpu.sync_copy(f2.at[pl.ds(c0, CHUNK)], f2rows)
            pltpu.sync_copy(ev.at[pl.ds(c0, CHUNK)], evv.at[pl.ds(0, CHUNK)])

            def e_body(e, _):
                ev_e = evv[pl.ds(e, LANES)][0]
                for hh in range(H // LANES):
                    s = pl.ds(hh * LANES, LANES)
                    rows[e, s] = rows[e, s] * f1rows[e, s] + ev_e * f2rows[e, s]
                return 0

            lax.fori_loop(0, CHUNK, e_body, 0)
            pltpu.sync_copy(rows, acc.at[didx], add=True)
            return 0

        lax.fori_loop(0, _num_chunks(wid), chunk_body, 0)
        plsc.subcore_barrier()
        pltpu.sync_copy(acc.at[pl.ds(r0, ROWS_PER_TILE)],
                        out.at[cid, pl.ds(r0, ROWS_PER_TILE)])

    return vector_pass


_scalar_pass = _make_scalar_pass()
_vector_pass = _make_vector_pass()


def kernel(scalar_features, vector_features, edge_index, edge_rbf,
           edge_vector, W1, b1, W2, b2):
    src = edge_index[0]
    dst = edge_index[1]
    evT = edge_vector.T
    vfT = jnp.transpose(vector_features, (1, 0, 2))

    fs, f1, f2 = _edge_mlp(edge_rbf, W1, b1, W2, b2)
    zz = jnp.zeros((NPAD, H), jnp.float32)

    ps = _scalar_pass(scalar_features, src, dst, fs, zz)
    pv = [_vector_pass(vfT[c], src, dst, f1, f2, evT[c], zz)
          for c in range(3)]

    return _combine(scalar_features, vector_features, ps, pv[0], pv[1], pv[2])


# speedup vs baseline: 13.7209x; 1.0413x over previous
import functools

import jax
import jax.numpy as jnp
from jax import lax
from jax.experimental import pallas as pl
from jax.experimental.pallas import tpu as pltpu
from jax.experimental.pallas import tpu_sc as plsc

N = 10000
E = 160000
H = 128
NRBF = 20
LANES = 16

CHUNK = 128
NCHUNKS = E // CHUNK
NW = 32
NTILES = 16
NPAD = 10112
ROWS_PER_TILE = NPAD // NTILES

BE = 640
BN = 400


def _mlp_body(rbf_ref, ev_ref, w1_ref, b1_ref, w2_ref, b2_ref,
              fs_ref, f1_ref, t0_ref, t1_ref, t2_ref):
    x = jnp.dot(rbf_ref[...], w1_ref[...],
                preferred_element_type=jnp.float32) + b1_ref[...]
    h = x * jax.nn.sigmoid(x)
    fw = jnp.dot(h, w2_ref[...], preferred_element_type=jnp.float32)
    fw = fw + b2_ref[...]
    fs_ref[...] = fw[:, 0:H]
    f1_ref[...] = fw[:, H:2 * H]
    f2 = fw[:, 2 * H:3 * H]
    t0_ref[...] = f2 * ev_ref[:, 0:1]
    t1_ref[...] = f2 * ev_ref[:, 1:2]
    t2_ref[...] = f2 * ev_ref[:, 2:3]


def _edge_mlp(rbf, ev, W1, b1, W2, b2):
    out = jax.ShapeDtypeStruct((E, H), jnp.float32)
    return pl.pallas_call(
        _mlp_body,
        grid=(E // BE,),
        in_specs=[
            pl.BlockSpec((BE, NRBF), lambda i: (i, 0)),
            pl.BlockSpec((BE, 3), lambda i: (i, 0)),
            pl.BlockSpec((NRBF, H), lambda i: (0, 0)),
            pl.BlockSpec((1, H), lambda i: (0, 0)),
            pl.BlockSpec((H, 3 * H), lambda i: (0, 0)),
            pl.BlockSpec((1, 3 * H), lambda i: (0, 0)),
        ],
        out_specs=[pl.BlockSpec((BE, H), lambda i: (i, 0))] * 5,
        out_shape=[out, out, out, out, out],
    )(rbf, ev, W1, b1.reshape(1, H), W2, b2.reshape(1, 3 * H))


def _combine_body(sf_ref, vf_ref, ps_ref, p0_ref, p1_ref, p2_ref,
                  so_ref, vo_ref):
    so_ref[...] = sf_ref[...] + ps_ref[0] + ps_ref[1]
    vo_ref[:, 0, :] = vf_ref[:, 0, :] + p0_ref[0] + p0_ref[1]
    vo_ref[:, 1, :] = vf_ref[:, 1, :] + p1_ref[0] + p1_ref[1]
    vo_ref[:, 2, :] = vf_ref[:, 2, :] + p2_ref[0] + p2_ref[1]


def _combine(sf, vf, ps, p0, p1, p2):
    part_spec = pl.BlockSpec((2, BN, H), lambda i: (0, i, 0))
    return pl.pallas_call(
        _combine_body,
        grid=(N // BN,),
        in_specs=[
            pl.BlockSpec((BN, H), lambda i: (i, 0)),
            pl.BlockSpec((BN, 3, H), lambda i: (i, 0, 0)),
            part_spec, part_spec, part_spec, part_spec,
        ],
        out_specs=[
            pl.BlockSpec((BN, H), lambda i: (i, 0)),
            pl.BlockSpec((BN, 3, H), lambda i: (i, 0, 0)),
        ],
        out_shape=[
            jax.ShapeDtypeStruct((N, H), jnp.float32),
            jax.ShapeDtypeStruct((N, 3, H), jnp.float32),
        ],
    )(sf, vf, ps, p0, p1, p2)


def _num_chunks(wid):
    base = NCHUNKS // NW
    return lax.select(wid < NCHUNKS % NW, base + 1, base)


def _make_scalar_pass():
    mesh = plsc.VectorSubcoreMesh(core_axis_name="c", subcore_axis_name="s")

    @functools.partial(
        pl.kernel, mesh=mesh,
        out_type=jax.ShapeDtypeStruct((2, NPAD, H), jnp.float32),
        scratch_types=[
            pltpu.VMEM((CHUNK,), jnp.int32),
            pltpu.VMEM((CHUNK,), jnp.int32),
            pltpu.VMEM((CHUNK, H), jnp.float32),
            pltpu.VMEM((CHUNK, H), jnp.float32),
            pltpu.VMEM_SHARED((NPAD, H), jnp.float32),
            pltpu.SemaphoreType.DMA,
        ],
    )
    def scalar_pass(tbl, srci, dsti, fs, zz, out, sidx, didx, rows, frows,
                    acc, sem):
        cid = lax.axis_index("c")
        sid = lax.axis_index("s")
        wid = sid * 2 + cid
        r0 = sid * ROWS_PER_TILE
        pltpu.sync_copy(zz.at[pl.ds(r0, ROWS_PER_TILE)],
                        acc.at[pl.ds(r0, ROWS_PER_TILE)])
        plsc.subcore_barrier()

        def chunk_body(i, carry):
            c0 = (wid + i * NW) * CHUNK
            pltpu.sync_copy(srci.at[pl.ds(c0, CHUNK)], sidx)
            pltpu.sync_copy(dsti.at[pl.ds(c0, CHUNK)], didx)
            cp = pltpu.async_copy(tbl.at[sidx], rows, sem)
            pltpu.sync_copy(fs.at[pl.ds(c0, CHUNK)], frows)
            cp.wait()

            def e_body(e, _):
                for hh in range(H // LANES):
                    s = pl.ds(hh * LANES, LANES)
                    rows[e, s] = rows[e, s] * frows[e, s]
                return 0

            lax.fori_loop(0, CHUNK, e_body, 0)
            pltpu.sync_copy(rows, acc.at[didx], add=True)
            return 0

        lax.fori_loop(0, _num_chunks(wid), chunk_body, 0)
        plsc.subcore_barrier()
        pltpu.sync_copy(acc.at[pl.ds(r0, ROWS_PER_TILE)],
                        out.at[cid, pl.ds(r0, ROWS_PER_TILE)])

    return scalar_pass


def _make_vector_pass():
    mesh = plsc.VectorSubcoreMesh(core_axis_name="c", subcore_axis_name="s")

    @functools.partial(
        pl.kernel, mesh=mesh,
        out_type=jax.ShapeDtypeStruct((2, NPAD, H), jnp.float32),
        scratch_types=[
            pltpu.VMEM((CHUNK,), jnp.int32),
            pltpu.VMEM((CHUNK,), jnp.int32),
            pltpu.VMEM((CHUNK, H), jnp.float32),
            pltpu.VMEM((CHUNK, H), jnp.float32),
            pltpu.VMEM((CHUNK, H), jnp.float32),
            pltpu.VMEM_SHARED((NPAD, H), jnp.float32),
            pltpu.SemaphoreType.DMA,
        ],
    )
    def vector_pass(tbl, srci, dsti, f1, t2, zz, out, sidx, didx, rows,
                    f1rows, t2rows, acc, sem):
        cid = lax.axis_index("c")
        sid = lax.axis_index("s")
        wid = sid * 2 + cid
        r0 = sid * ROWS_PER_TILE
        pltpu.sync_copy(zz.at[pl.ds(r0, ROWS_PER_TILE)],
                        acc.at[pl.ds(r0, ROWS_PER_TILE)])
        plsc.subcore_barrier()

        def chunk_body(i, carry):
            c0 = (wid + i * NW) * CHUNK
            pltpu.sync_copy(srci.at[pl.ds(c0, CHUNK)], sidx)
            pltpu.sync_copy(dsti.at[pl.ds(c0, CHUNK)], didx)
            cp = pltpu.async_copy(tbl.at[sidx], rows, sem)
            pltpu.sync_copy(f1.at[pl.ds(c0, CHUNK)], f1rows)
            pltpu.sync_copy(t2.at[pl.ds(c0, CHUNK)], t2rows)
            pltpu.sync_copy(t2rows, acc.at[didx], add=True)
            cp.wait()

            def e_body(e, _):
                for hh in range(H // LANES):
                    s = pl.ds(hh * LANES, LANES)
                    rows[e, s] = rows[e, s] * f1rows[e, s]
                return 0

            lax.fori_loop(0, CHUNK, e_body, 0)
            pltpu.sync_copy(rows, acc.at[didx], add=True)
            return 0

        lax.fori_loop(0, _num_chunks(wid), chunk_body, 0)
        plsc.subcore_barrier()
        pltpu.sync_copy(acc.at[pl.ds(r0, ROWS_PER_TILE)],
                        out.at[cid, pl.ds(r0, ROWS_PER_TILE)])

    return vector_pass


_scalar_pass = _make_scalar_pass()
_vector_pass = _make_vector_pass()


def kernel(scalar_features, vector_features, edge_index, edge_rbf,
           edge_vector, W1, b1, W2, b2):
    src = edge_index[0]
    dst = edge_index[1]
    vfT = jnp.transpose(vector_features, (1, 0, 2))

    fs, f1, t0, t1, t2 = _edge_mlp(edge_rbf, edge_vector, W1, b1, W2, b2)
    zz = jnp.zeros((NPAD, H), jnp.float32)

    ps = _scalar_pass(scalar_features, src, dst, fs, zz)
    pv = [_vector_pass(vfT[c], src, dst, f1, t, zz)
          for c, t in enumerate((t0, t1, t2))]

    return _combine(scalar_features, vector_features, ps, pv[0], pv[1], pv[2])


# speedup vs baseline: 13.8728x; 1.0111x over previous
import functools

import jax
import jax.numpy as jnp
from jax import lax
from jax.experimental import pallas as pl
from jax.experimental.pallas import tpu as pltpu
from jax.experimental.pallas import tpu_sc as plsc

N = 10000
E = 160000
H = 128
NRBF = 20
LANES = 16

CHUNK = 128
NCHUNKS = E // CHUNK
NW = 32
NTILES = 16
NPAD = 10112
ROWS_PER_TILE = NPAD // NTILES

BE = 640
BN = 400


def _mlp_body(rbf_ref, ev_ref, w1_ref, b1_ref, w2_ref, b2_ref,
              fs_ref, f1_ref, t0_ref, t1_ref, t2_ref):
    x = jnp.dot(rbf_ref[...], w1_ref[...],
                preferred_element_type=jnp.float32) + b1_ref[...]
    h = x * jax.nn.sigmoid(x)
    fw = jnp.dot(h, w2_ref[...], preferred_element_type=jnp.float32)
    fw = fw + b2_ref[...]
    fs_ref[...] = fw[:, 0:H]
    f1_ref[...] = fw[:, H:2 * H]
    f2 = fw[:, 2 * H:3 * H]
    t0_ref[...] = f2 * ev_ref[:, 0:1]
    t1_ref[...] = f2 * ev_ref[:, 1:2]
    t2_ref[...] = f2 * ev_ref[:, 2:3]


def _edge_mlp(rbf, ev, W1, b1, W2, b2):
    out = jax.ShapeDtypeStruct((E, H), jnp.float32)
    return pl.pallas_call(
        _mlp_body,
        grid=(E // BE,),
        in_specs=[
            pl.BlockSpec((BE, NRBF), lambda i: (i, 0)),
            pl.BlockSpec((BE, 3), lambda i: (i, 0)),
            pl.BlockSpec((NRBF, H), lambda i: (0, 0)),
            pl.BlockSpec((1, H), lambda i: (0, 0)),
            pl.BlockSpec((H, 3 * H), lambda i: (0, 0)),
            pl.BlockSpec((1, 3 * H), lambda i: (0, 0)),
        ],
        out_specs=[pl.BlockSpec((BE, H), lambda i: (i, 0))] * 5,
        out_shape=[out, out, out, out, out],
    )(rbf, ev, W1, b1.reshape(1, H), W2, b2.reshape(1, 3 * H))


def _combine_body(sf_ref, vf_ref, ps_ref, p0_ref, p1_ref, p2_ref,
                  so_ref, vo_ref):
    so_ref[...] = sf_ref[...] + ps_ref[0] + ps_ref[1]
    vo_ref[:, 0, :] = vf_ref[:, 0, :] + p0_ref[0] + p0_ref[1]
    vo_ref[:, 1, :] = vf_ref[:, 1, :] + p1_ref[0] + p1_ref[1]
    vo_ref[:, 2, :] = vf_ref[:, 2, :] + p2_ref[0] + p2_ref[1]


def _combine(sf, vf, ps, p0, p1, p2):
    part_spec = pl.BlockSpec((2, BN, H), lambda i: (0, i, 0))
    return pl.pallas_call(
        _combine_body,
        grid=(N // BN,),
        in_specs=[
            pl.BlockSpec((BN, H), lambda i: (i, 0)),
            pl.BlockSpec((BN, 3, H), lambda i: (i, 0, 0)),
            part_spec, part_spec, part_spec, part_spec,
        ],
        out_specs=[
            pl.BlockSpec((BN, H), lambda i: (i, 0)),
            pl.BlockSpec((BN, 3, H), lambda i: (i, 0, 0)),
        ],
        out_shape=[
            jax.ShapeDtypeStruct((N, H), jnp.float32),
            jax.ShapeDtypeStruct((N, 3, H), jnp.float32),
        ],
    )(sf, vf, ps, p0, p1, p2)


def _num_chunks(wid):
    base = NCHUNKS // NW
    return lax.select(wid < NCHUNKS % NW, base + 1, base)


def _make_scalar_pass():
    mesh = plsc.VectorSubcoreMesh(core_axis_name="c", subcore_axis_name="s")

    @functools.partial(
        pl.kernel, mesh=mesh,
        out_type=jax.ShapeDtypeStruct((2, NPAD, H), jnp.float32),
        scratch_types=[
            pltpu.VMEM((CHUNK,), jnp.int32),
            pltpu.VMEM((CHUNK,), jnp.int32),
            pltpu.VMEM((CHUNK, H), jnp.float32),
            pltpu.VMEM((CHUNK, H), jnp.float32),
            pltpu.VMEM_SHARED((NPAD, H), jnp.float32),
            pltpu.SemaphoreType.DMA,
        ],
    )
    def scalar_pass(tbl, srci, dsti, fs, zz, out, sidx, didx, rows, frows,
                    acc, sem):
        cid = lax.axis_index("c")
        sid = lax.axis_index("s")
        wid = sid * 2 + cid
        r0 = sid * ROWS_PER_TILE
        pltpu.sync_copy(zz.at[pl.ds(r0, ROWS_PER_TILE)],
                        acc.at[pl.ds(r0, ROWS_PER_TILE)])
        plsc.subcore_barrier()

        def chunk_body(i, carry):
            c0 = (wid + i * NW) * CHUNK
            pltpu.sync_copy(srci.at[pl.ds(c0, CHUNK)], sidx)
            pltpu.sync_copy(dsti.at[pl.ds(c0, CHUNK)], didx)
            cp = pltpu.async_copy(tbl.at[sidx], rows, sem)
            pltpu.sync_copy(fs.at[pl.ds(c0, CHUNK)], frows)
            cp.wait()

            def e_body(e, _):
                for hh in range(H // LANES):
                    s = pl.ds(hh * LANES, LANES)
                    rows[e, s] = rows[e, s] * frows[e, s]
                return 0

            lax.fori_loop(0, CHUNK, e_body, 0)
            pltpu.sync_copy(rows, acc.at[didx], add=True)
            return 0

        lax.fori_loop(0, _num_chunks(wid), chunk_body, 0)
        plsc.subcore_barrier()
        pltpu.sync_copy(acc.at[pl.ds(r0, ROWS_PER_TILE)],
                        out.at[cid, pl.ds(r0, ROWS_PER_TILE)])

    return scalar_pass


def _make_vector_pass():
    mesh = plsc.VectorSubcoreMesh(core_axis_name="c", subcore_axis_name="s")

    @functools.partial(
        pl.kernel, mesh=mesh,
        out_type=jax.ShapeDtypeStruct((2, NPAD, H), jnp.float32),
        scratch_types=[
            pltpu.VMEM((CHUNK,), jnp.int32),
            pltpu.VMEM((CHUNK,), jnp.int32),
            pltpu.VMEM((CHUNK, H), jnp.float32),
            pltpu.VMEM((CHUNK, H), jnp.float32),
            pltpu.VMEM((CHUNK, H), jnp.float32),
            pltpu.VMEM_SHARED((NPAD, H), jnp.float32),
            pltpu.SemaphoreType.DMA,
        ],
    )
    def vector_pass(tbl, srci, dsti, f1, t2, zz, out, sidx, didx, rows,
                    f1rows, t2rows, acc, sem):
        cid = lax.axis_index("c")
        sid = lax.axis_index("s")
        wid = sid * 2 + cid
        r0 = sid * ROWS_PER_TILE
        pltpu.sync_copy(zz.at[pl.ds(r0, ROWS_PER_TILE)],
                        acc.at[pl.ds(r0, ROWS_PER_TILE)])
        plsc.subcore_barrier()

        def chunk_body(i, carry):
            c0 = (wid + i * NW) * CHUNK
            pltpu.sync_copy(srci.at[pl.ds(c0, CHUNK)], sidx)
            pltpu.sync_copy(dsti.at[pl.ds(c0, CHUNK)], didx)
            cp = pltpu.async_copy(tbl.at[sidx], rows, sem)
            pltpu.sync_copy(f1.at[pl.ds(c0, CHUNK)], f1rows)
            pltpu.sync_copy(t2.at[pl.ds(c0, CHUNK)], t2rows)
            cp.wait()

            def e_body(e, _):
                for hh in range(H // LANES):
                    s = pl.ds(hh * LANES, LANES)
                    rows[e, s] = rows[e, s] * f1rows[e, s] + t2rows[e, s]
                return 0

            lax.fori_loop(0, CHUNK, e_body, 0)
            pltpu.sync_copy(rows, acc.at[didx], add=True)
            return 0

        lax.fori_loop(0, _num_chunks(wid), chunk_body, 0)
        plsc.subcore_barrier()
        pltpu.sync_copy(acc.at[pl.ds(r0, ROWS_PER_TILE)],
                        out.at[cid, pl.ds(r0, ROWS_PER_TILE)])

    return vector_pass


_scalar_pass = _make_scalar_pass()
_vector_pass = _make_vector_pass()


def kernel(scalar_features, vector_features, edge_index, edge_rbf,
           edge_vector, W1, b1, W2, b2):
    src = edge_index[0]
    dst = edge_index[1]
    vfT = jnp.transpose(vector_features, (1, 0, 2))

    fs, f1, t0, t1, t2 = _edge_mlp(edge_rbf, edge_vector, W1, b1, W2, b2)
    zz = jnp.zeros((NPAD, H), jnp.float32)

    ps = _scalar_pass(scalar_features, src, dst, fs, zz)
    pv = [_vector_pass(vfT[c], src, dst, f1, t, zz)
          for c, t in enumerate((t0, t1, t2))]

    return _combine(scalar_features, vector_features, ps, pv[0], pv[1], pv[2])


# speedup vs baseline: 14.1049x; 1.0167x over previous
import functools

import jax
import jax.numpy as jnp
from jax import lax
from jax.experimental import pallas as pl
from jax.experimental.pallas import tpu as pltpu
from jax.experimental.pallas import tpu_sc as plsc

N = 10000
E = 160000
H = 128
NRBF = 20
LANES = 16

CHUNK = 128
NCHUNKS = E // CHUNK
NW = 32
NTILES = 16
NPAD = 10112
ROWS_PER_TILE = NPAD // NTILES

BE = 640
BN = 400


def _mlp_body(rbf_ref, ev_ref, w1_ref, b1_ref, w2_ref, b2_ref,
              fs_ref, f1_ref, t0_ref, t1_ref, t2_ref):
    x = jnp.dot(rbf_ref[...], w1_ref[...],
                preferred_element_type=jnp.float32) + b1_ref[...]
    h = x * jax.nn.sigmoid(x)
    fw = jnp.dot(h, w2_ref[...], preferred_element_type=jnp.float32)
    fw = fw + b2_ref[...]
    fs_ref[...] = fw[:, 0:H]
    f1_ref[...] = fw[:, H:2 * H]
    f2 = fw[:, 2 * H:3 * H]
    t0_ref[...] = f2 * ev_ref[:, 0:1]
    t1_ref[...] = f2 * ev_ref[:, 1:2]
    t2_ref[...] = f2 * ev_ref[:, 2:3]


def _edge_mlp(rbf, ev, W1, b1, W2, b2):
    out = jax.ShapeDtypeStruct((E, H), jnp.float32)
    return pl.pallas_call(
        _mlp_body,
        grid=(E // BE,),
        in_specs=[
            pl.BlockSpec((BE, NRBF), lambda i: (i, 0)),
            pl.BlockSpec((BE, 3), lambda i: (i, 0)),
            pl.BlockSpec((NRBF, H), lambda i: (0, 0)),
            pl.BlockSpec((1, H), lambda i: (0, 0)),
            pl.BlockSpec((H, 3 * H), lambda i: (0, 0)),
            pl.BlockSpec((1, 3 * H), lambda i: (0, 0)),
        ],
        out_specs=[pl.BlockSpec((BE, H), lambda i: (i, 0))] * 5,
        out_shape=[out, out, out, out, out],
    )(rbf, ev, W1, b1.reshape(1, H), W2, b2.reshape(1, 3 * H))


def _combine_body(sf_ref, vf_ref, p_ref, so_ref, vo_ref):
    so_ref[...] = sf_ref[...] + p_ref[0, 0] + p_ref[0, 1]
    vo_ref[:, 0, :] = vf_ref[:, 0, :] + p_ref[1, 0] + p_ref[1, 1]
    vo_ref[:, 1, :] = vf_ref[:, 1, :] + p_ref[2, 0] + p_ref[2, 1]
    vo_ref[:, 2, :] = vf_ref[:, 2, :] + p_ref[3, 0] + p_ref[3, 1]


def _combine(sf, vf, parts):
    return pl.pallas_call(
        _combine_body,
        grid=(N // BN,),
        in_specs=[
            pl.BlockSpec((BN, H), lambda i: (i, 0)),
            pl.BlockSpec((BN, 3, H), lambda i: (i, 0, 0)),
            pl.BlockSpec((4, 2, BN, H), lambda i: (0, 0, i, 0)),
        ],
        out_specs=[
            pl.BlockSpec((BN, H), lambda i: (i, 0)),
            pl.BlockSpec((BN, 3, H), lambda i: (i, 0, 0)),
        ],
        out_shape=[
            jax.ShapeDtypeStruct((N, H), jnp.float32),
            jax.ShapeDtypeStruct((N, 3, H), jnp.float32),
        ],
    )(sf, vf, parts)


def _num_chunks(wid):
    base = NCHUNKS // NW
    return lax.select(wid < NCHUNKS % NW, base + 1, base)


def _make_sc_pass():
    mesh = plsc.VectorSubcoreMesh(core_axis_name="c", subcore_axis_name="s")

    @functools.partial(
        pl.kernel, mesh=mesh,
        out_type=jax.ShapeDtypeStruct((4, 2, NPAD, H), jnp.float32),
        scratch_types=[
            pltpu.VMEM((CHUNK,), jnp.int32),
            pltpu.VMEM((CHUNK,), jnp.int32),
            pltpu.VMEM((CHUNK, H), jnp.float32),
            pltpu.VMEM((CHUNK, H), jnp.float32),
            pltpu.VMEM((CHUNK, H), jnp.float32),
            pltpu.VMEM_SHARED((NPAD, H), jnp.float32),
            pltpu.SemaphoreType.DMA,
        ],
    )
    def sc_pass(sf, v0, v1, v2, srci, dsti, fs, f1, t0, t1, t2, zz, out,
                sidx, didx, rows, f1rows, t2rows, acc, sem):
        cid = lax.axis_index("c")
        sid = lax.axis_index("s")
        wid = sid * 2 + cid
        r0 = sid * ROWS_PER_TILE

        phases = ((sf, fs, None), (v0, f1, t0), (v1, f1, t1), (v2, f1, t2))
        for p, (tbl, fil, t2f) in enumerate(phases):
            pltpu.sync_copy(zz.at[pl.ds(r0, ROWS_PER_TILE)],
                            acc.at[pl.ds(r0, ROWS_PER_TILE)])
            plsc.subcore_barrier()

            def chunk_body(i, carry):
                c0 = (wid + i * NW) * CHUNK
                pltpu.sync_copy(srci.at[pl.ds(c0, CHUNK)], sidx)
                pltpu.sync_copy(dsti.at[pl.ds(c0, CHUNK)], didx)
                cp = pltpu.async_copy(tbl.at[sidx], rows, sem)
                pltpu.sync_copy(fil.at[pl.ds(c0, CHUNK)], f1rows)
                if t2f is not None:
                    pltpu.sync_copy(t2f.at[pl.ds(c0, CHUNK)], t2rows)
                cp.wait()

                if t2f is None:
                    def e_body(e, _):
                        for hh in range(H // LANES):
                            s = pl.ds(hh * LANES, LANES)
                            rows[e, s] = rows[e, s] * f1rows[e, s]
                        return 0
                else:
                    def e_body(e, _):
                        for hh in range(H // LANES):
                            s = pl.ds(hh * LANES, LANES)
                            rows[e, s] = (rows[e, s] * f1rows[e, s]
                                          + t2rows[e, s])
                        return 0

                lax.fori_loop(0, CHUNK, e_body, 0)
                pltpu.sync_copy(rows, acc.at[didx], add=True)
                return 0

            lax.fori_loop(0, _num_chunks(wid), chunk_body, 0)
            plsc.subcore_barrier()
            pltpu.sync_copy(acc.at[pl.ds(r0, ROWS_PER_TILE)],
                            out.at[p, cid, pl.ds(r0, ROWS_PER_TILE)])

    return sc_pass


_sc_pass = _make_sc_pass()


def kernel(scalar_features, vector_features, edge_index, edge_rbf,
           edge_vector, W1, b1, W2, b2):
    src = edge_index[0]
    dst = edge_index[1]
    vfT = jnp.transpose(vector_features, (1, 0, 2))

    fs, f1, t0, t1, t2 = _edge_mlp(edge_rbf, edge_vector, W1, b1, W2, b2)
    zz = jnp.zeros((NPAD, H), jnp.float32)

    parts = _sc_pass(scalar_features, vfT[0], vfT[1], vfT[2],
                     src, dst, fs, f1, t0, t1, t2, zz)

    return _combine(scalar_features, vector_features, parts)
